# baseline (speedup 1.0000x reference)
import jax
import jax.numpy as jnp
from jax.experimental import pallas as pl

_BINS = 64
_EDGES = _BINS + 1
_EPAD = 72
_LANES = 128
_HW = 384 * 384
_ROWS = _HW // _LANES
_PLANES = 6
_NT = 2 * _PLANES
_HALF_SD = 100.0 / (2 * _BINS)
_UNROLL = 8


def _acc_kernel(x_ref, acc_ref):
    dvec = _HALF_SD * jax.lax.broadcasted_iota(jnp.int32, (_EPAD, 1), 0).astype(jnp.float32)

    def body(k, accs):
        tile = x_ref[0, pl.ds(k * _UNROLL, _UNROLL), :] * 50.0
        new = []
        for u in range(_UNROLL):
            row = tile[u:u + 1, :]
            t = jnp.tanh(jnp.broadcast_to(row, (_EPAD, _LANES)) - dvec)
            new.append(accs[u % 2] + t if u < 2 else new[u - 2] + t)
        return (new[_UNROLL - 2], new[_UNROLL - 1])

    zero = jnp.zeros((_EPAD, _LANES), jnp.float32)
    accs = jax.lax.fori_loop(0, _ROWS // _UNROLL, body, (zero, zero))
    acc_ref[0] = accs[0] + accs[1]


def _loss_kernel(acc_ref, loss_ref):
    total = jnp.zeros((1, 1), jnp.float32)
    for p in range(_PLANES):
        t_o = jnp.sum(acc_ref[p], axis=1, keepdims=True)
        t_t = jnp.sum(acc_ref[p + _PLANES], axis=1, keepdims=True)
        d_o = t_o[0:_BINS] - t_o[1:_EDGES]
        d_t = t_t[0:_BINS] - t_t[1:_EDGES]
        total = total + jnp.full((1, 1), 0.5 * jnp.sum(jnp.abs(d_o - d_t)))
    loss_ref[...] = total * (1.0 / (_PLANES * _BINS * _HW))


@jax.jit
def kernel(output, target):
    o = output.reshape(_PLANES, _ROWS, _LANES)
    t = target.reshape(_PLANES, _ROWS, _LANES)
    x = jnp.concatenate([o, t], axis=0)
    acc = pl.pallas_call(
        _acc_kernel,
        grid=(_NT,),
        in_specs=[pl.BlockSpec((1, _ROWS, _LANES), lambda p: (p, 0, 0))],
        out_specs=pl.BlockSpec((1, _EPAD, _LANES), lambda p: (p, 0, 0)),
        out_shape=jax.ShapeDtypeStruct((_NT, _EPAD, _LANES), jnp.float32),
    )(x)
    loss = pl.pallas_call(
        _loss_kernel,
        out_shape=jax.ShapeDtypeStruct((1, 1), jnp.float32),
    )(acc)
    return loss[0, 0]
